# Initial kernel scaffold; baseline (speedup 1.0000x reference)
import jax
import jax.numpy as jnp
from jax.experimental import pallas as pl


def kernel(x, edge_index, W, att_src, att_dst, bias):
    raise NotImplementedError("write your pallas kernel here")


# speedup vs baseline: 10.8343x; 10.8343x over previous
import jax
import jax.numpy as jnp
from jax import lax
from jax.experimental import pallas as pl
from jax.experimental.pallas import tpu as pltpu
from jax.experimental.pallas import tpu_sc as plsc

D_MODEL = 128
N_HEADS = 4
OUT_CH = 128
HC = N_HEADS * OUT_CH
WROW = HC + OUT_CH
ASRC0 = HC + 8

N_NODES = 40000
E_TOT = 680000
N_TILES = 16
E_PAD = 688128
ET = E_PAD // N_TILES
CH = 1344
NV = CH // 16
R = 1920
NPASS = 11
STRIPE = R // N_TILES
LROWS = STRIPE + 8
DUMMYL = STRIPE
CBP = 5120
CB2 = 3072
G = 16


def _iota16():
    return lax.iota(jnp.int32, 16)


def _tc_body(x_ref, w_ref, asrc_ref, adst_ref, h_ref, ad_ref):
    hb = jnp.dot(x_ref[...], w_ref[...], preferred_element_type=jnp.float32)
    h_ref[:, :HC] = hb
    h_ref[:, HC:] = jnp.zeros((x_ref.shape[0], WROW - HC), jnp.float32)
    for hh in range(N_HEADS):
        blk = hb[:, hh * OUT_CH:(hh + 1) * OUT_CH]
        h_ref[:, ASRC0 + hh:ASRC0 + hh + 1] = jnp.sum(
            blk * asrc_ref[hh:hh + 1, :], axis=1, keepdims=True)
        ad_ref[:, hh:hh + 1] = jnp.sum(
            blk * adst_ref[hh:hh + 1, :], axis=1, keepdims=True)


def _tc_project(xf, W, att_src, att_dst):
    BM = 512
    grid = (N_NODES + BM - 1) // BM
    return pl.pallas_call(
        _tc_body,
        grid=(grid,),
        in_specs=[
            pl.BlockSpec((BM, D_MODEL), lambda i: (i, 0)),
            pl.BlockSpec((D_MODEL, HC), lambda i: (0, 0)),
            pl.BlockSpec((N_HEADS, OUT_CH), lambda i: (0, 0)),
            pl.BlockSpec((N_HEADS, OUT_CH), lambda i: (0, 0)),
        ],
        out_specs=[
            pl.BlockSpec((BM, WROW), lambda i: (i, 0)),
            pl.BlockSpec((BM, N_HEADS), lambda i: (i, 0)),
        ],
        out_shape=[
            jax.ShapeDtypeStruct((N_NODES, WROW), jnp.float32),
            jax.ShapeDtypeStruct((N_NODES, N_HEADS), jnp.float32),
        ],
    )(xf, W, att_src, att_dst)


def _sc_body(h_hbm, ad_hbm, src_hbm, dst_hbm, bias_hbm, out_hbm,
             sd_src, sd_dst, csrcP, crelP, lbufS, lbufR, csrc2, crel2,
             adch, hrows, accL, nout, biasb, ssrc, srel):
    cid = lax.axis_index("c")
    sid = lax.axis_index("s")
    it = _iota16()
    zero16f = jnp.zeros((16,), jnp.float32)
    zero16i = jnp.zeros((16,), jnp.int32)

    pltpu.sync_copy(bias_hbm, biasb)

    def pass_body(p, _):
        lo = p * (2 * R) + cid * R

        def _zacc(r, _):
            for q in range(WROW // 16):
                accL[r, pl.ds(q * 16, 16)] = zero16f
            return 0
        lax.fori_loop(0, LROWS, _zacc, 0)

        def _pfP(r, _):
            for q in range(128 // 16):
                csrcP[r, pl.ds(q * 16, 16)] = zero16i
                crelP[r, pl.ds(q * 16, 16)] = jnp.full((16,), R, jnp.int32)
            return 0
        lax.fori_loop(0, CBP // 128, _pfP, 0)

        def chunk_body(ci, k):
            eoff = sid * ET + ci * CH
            pltpu.sync_copy(src_hbm.at[pl.ds(eoff, CH)], sd_src)
            pltpu.sync_copy(dst_hbm.at[pl.ds(eoff, CH)], sd_dst)

            def vreg_body(vi, k):
                dv = sd_dst[pl.ds(vi * 16, 16)]
                sv = sd_src[pl.ds(vi * 16, 16)]
                rel = dv - lo
                m = (rel >= 0) & (rel < R)
                mi = m.astype(jnp.int32)
                pos = k + plsc.cumsum(mi) - 1
                m = m & (pos < CBP)
                plsc.store_scatter(csrcP, [pos >> 7, pos & 127], sv, mask=m)
                plsc.store_scatter(crelP, [pos >> 7, pos & 127], rel, mask=m)
                return k + jnp.sum(mi)

            return lax.fori_loop(0, NV, vreg_body, k)

        lax.fori_loop(0, ET // CH, chunk_body, jnp.int32(0))

        pltpu.sync_copy(csrcP, ssrc.at[sid])
        pltpu.sync_copy(crelP, srel.at[sid])
        pltpu.sync_copy(
            ad_hbm.at[pl.ds((lo + sid * STRIPE) * N_HEADS,
                            LROWS * N_HEADS)], adch)
        plsc.subcore_barrier()

        def _pf2(i, _):
            csrc2[pl.ds(i * 16, 16)] = zero16i
            crel2[pl.ds(i * 16, 16)] = jnp.full((16,), DUMMYL, jnp.int32)
            return 0
        lax.fori_loop(0, CB2 // 16, _pf2, 0)

        slo = sid * STRIPE

        def t_body(tp, k2):
            def c_body(c, k2):
                pltpu.sync_copy(ssrc.at[tp, pl.ds(c * 8, 8)], lbufS)
                pltpu.sync_copy(srel.at[tp, pl.ds(c * 8, 8)], lbufR)

                def v_body(v, k2):
                    row = v >> 3
                    colb = (v & 7) * 16
                    relv = lbufR[row, pl.ds(colb, 16)]
                    sv = lbufS[row, pl.ds(colb, 16)]
                    rloc = relv - slo
                    m = (rloc >= 0) & (rloc < STRIPE)
                    mi = m.astype(jnp.int32)
                    pos = k2 + plsc.cumsum(mi) - 1
                    m = m & (pos < CB2)
                    plsc.store_scatter(csrc2, [pos], sv, mask=m)
                    plsc.store_scatter(crel2, [pos], rloc, mask=m)
                    return k2 + jnp.sum(mi)

                return lax.fori_loop(0, 64, v_body, k2)
            return lax.fori_loop(0, CBP // 1024, c_body, k2)

        k2 = lax.fori_loop(0, N_TILES, t_body, jnp.int32(0))
        k2 = jnp.minimum(k2, CB2)

        def group_body(g, _):
            pltpu.sync_copy(h_hbm.at[csrc2.at[pl.ds(g * G, G)]], hrows)

            def scale_acc(e, _):
                es = jnp.full((16,), e, jnp.int32)
                rel = plsc.load_gather(crel2, [g * G + es])
                s = plsc.load_gather(hrows, [es, ASRC0 + (it & 3)])
                d = plsc.load_gather(adch, [rel * N_HEADS + (it & 3)])
                a = s + d
                pv = jnp.exp(jnp.maximum(a, 0.2 * a))
                hrows[e, pl.ds(HC, 16)] = pv
                plsc.addupdate_scatter(accL, [rel, HC + it], pv)
                for hh in range(N_HEADS):
                    sp = plsc.load_gather(
                        hrows, [es, jnp.full((16,), HC + hh, jnp.int32)])
                    for q in range(OUT_CH // 16):
                        col = hh * OUT_CH + q * 16
                        v = hrows[e, pl.ds(col, 16)] * sp
                        plsc.addupdate_scatter(accL, [rel, col + it], v)
                return 0
            lax.fori_loop(0, G, scale_acc, 0)
            return 0

        lax.fori_loop(0, (k2 + G - 1) // G, group_body, 0)

        def norm_body(t, _):
            base = lo + slo + t * 8

            @pl.when(base < N_NODES)
            def _():
                def row_body(j, _):
                    jr = t * 8 + j
                    js = jnp.full((16,), jr, jnp.int32)
                    sp = []
                    for hh in range(N_HEADS):
                        denom = plsc.load_gather(
                            accL, [js, jnp.full((16,), HC + hh, jnp.int32)])
                        sp.append(1.0 / (denom + 1e-16))
                    for q in range(OUT_CH // 16):
                        acc_v = accL[jr, pl.ds(q * 16, 16)] * sp[0]
                        for hh in range(1, N_HEADS):
                            acc_v = acc_v + (
                                accL[jr, pl.ds(hh * OUT_CH + q * 16, 16)]
                                * sp[hh])
                        nout[j, pl.ds(q * 16, 16)] = (
                            acc_v * 0.25 + biasb[pl.ds(q * 16, 16)])
                    return 0
                lax.fori_loop(0, 8, row_body, 0)

                pltpu.sync_copy(nout, out_hbm.at[pl.ds(base, 8)])
            return 0
        lax.fori_loop(0, STRIPE // 8, norm_body, 0)
        plsc.subcore_barrier()
        return 0

    lax.fori_loop(0, NPASS, pass_body, 0)


def _sc_aggregate(h, ad, src, dst, bias):
    mesh = plsc.VectorSubcoreMesh(core_axis_name="c", subcore_axis_name="s")
    f32 = jnp.float32
    i32 = jnp.int32
    kern = pl.kernel(
        _sc_body,
        out_type=jax.ShapeDtypeStruct((N_NODES, OUT_CH), f32),
        mesh=mesh,
        compiler_params=pltpu.CompilerParams(needs_layout_passes=False),
        scratch_types=[
            pltpu.VMEM((CH,), i32),
            pltpu.VMEM((CH,), i32),
            pltpu.VMEM((CBP // 128, 128), i32),
            pltpu.VMEM((CBP // 128, 128), i32),
            pltpu.VMEM((8, 128), i32),
            pltpu.VMEM((8, 128), i32),
            pltpu.VMEM((CB2,), i32),
            pltpu.VMEM((CB2,), i32),
            pltpu.VMEM((LROWS * N_HEADS,), f32),
            pltpu.VMEM((G, WROW), f32),
            pltpu.VMEM((LROWS, WROW), f32),
            pltpu.VMEM((8, OUT_CH), f32),
            pltpu.VMEM((OUT_CH,), f32),
            pltpu.VMEM_SHARED((N_TILES, CBP // 128, 128), i32),
            pltpu.VMEM_SHARED((N_TILES, CBP // 128, 128), i32),
        ],
    )
    return kern(h, ad, src, dst, bias)


def kernel(x, edge_index, W, att_src, att_dst, bias):
    b, s, n, f = x.shape
    xf = x.reshape(b * s * n, f)

    ei = edge_index.astype(jnp.int32)
    offs = jnp.arange(b * s, dtype=jnp.int32) * n
    src_r = (ei[0][None, :] + offs[:, None]).reshape(-1)
    dst_r = (ei[1][None, :] + offs[:, None]).reshape(-1)
    loop = jnp.arange(N_NODES, dtype=jnp.int32)
    pad = E_PAD - E_TOT
    src_all = jnp.concatenate(
        [src_r, loop, jnp.zeros((pad,), jnp.int32)])
    dst_all = jnp.concatenate(
        [dst_r, loop, jnp.full((pad,), jnp.int32(1 << 20))])
    src_all = src_all.reshape(ET, N_TILES).T.reshape(-1)
    dst_all = dst_all.reshape(ET, N_TILES).T.reshape(-1)

    h, ad = _tc_project(xf, W, att_src, att_dst)
    ad1 = jnp.pad(ad.reshape(-1),
                  (0, (NPASS * 2 * R + LROWS - N_NODES) * N_HEADS))
    out = _sc_aggregate(h, ad1, src_all, dst_all, bias)
    return out.reshape(b, s, n, OUT_CH)


# speedup vs baseline: 13.0299x; 1.2027x over previous
import jax
import jax.numpy as jnp
from jax import lax
from jax.experimental import pallas as pl
from jax.experimental.pallas import tpu as pltpu
from jax.experimental.pallas import tpu_sc as plsc

D_MODEL = 128
N_HEADS = 4
OUT_CH = 128
HC = N_HEADS * OUT_CH
WROW = HC + OUT_CH
ASRC0 = HC + 8

N_NODES = 40000
E_TOT = 680000
N_TILES = 16
E_PAD = 688128
ET = E_PAD // N_TILES
CH = 672
NV = CH // 16
R = 1920
NPASS = 11
STRIPE = R // N_TILES
LROWS = STRIPE + 8
DUMMYL = STRIPE
CBP = 4096
CB2 = 3072
G = 16


def _iota16():
    return lax.iota(jnp.int32, 16)


def _tc_body(x_ref, w_ref, asrc_ref, adst_ref, h_ref, ad_ref):
    hb = jnp.dot(x_ref[...], w_ref[...], preferred_element_type=jnp.float32)
    h_ref[:, :HC] = hb
    h_ref[:, HC:] = jnp.zeros((x_ref.shape[0], WROW - HC), jnp.float32)
    for hh in range(N_HEADS):
        blk = hb[:, hh * OUT_CH:(hh + 1) * OUT_CH]
        h_ref[:, ASRC0 + hh:ASRC0 + hh + 1] = jnp.sum(
            blk * asrc_ref[hh:hh + 1, :], axis=1, keepdims=True)
        ad_ref[:, hh:hh + 1] = jnp.sum(
            blk * adst_ref[hh:hh + 1, :], axis=1, keepdims=True)


def _tc_project(xf, W, att_src, att_dst):
    BM = 512
    grid = (N_NODES + BM - 1) // BM
    return pl.pallas_call(
        _tc_body,
        grid=(grid,),
        in_specs=[
            pl.BlockSpec((BM, D_MODEL), lambda i: (i, 0)),
            pl.BlockSpec((D_MODEL, HC), lambda i: (0, 0)),
            pl.BlockSpec((N_HEADS, OUT_CH), lambda i: (0, 0)),
            pl.BlockSpec((N_HEADS, OUT_CH), lambda i: (0, 0)),
        ],
        out_specs=[
            pl.BlockSpec((BM, WROW), lambda i: (i, 0)),
            pl.BlockSpec((BM, N_HEADS), lambda i: (i, 0)),
        ],
        out_shape=[
            jax.ShapeDtypeStruct((N_NODES, WROW), jnp.float32),
            jax.ShapeDtypeStruct((N_NODES, N_HEADS), jnp.float32),
        ],
    )(xf, W, att_src, att_dst)


def _sc_body(h_hbm, ad_hbm, src_hbm, dst_hbm, bias_hbm, out_hbm,
             sd_src, sd_dst, csrcP, crelP, lbufS, lbufR, csrc2, crel2,
             adch, hr0, hr1, accL, nout, biasb, ssrc, srel, sem0, sem1):
    cid = lax.axis_index("c")
    sid = lax.axis_index("s")
    it = _iota16()
    zero16f = jnp.zeros((16,), jnp.float32)
    zero16i = jnp.zeros((16,), jnp.int32)

    pltpu.sync_copy(bias_hbm, biasb)

    def pass_body(p, _):
        lo = p * (2 * R) + cid * R

        def _zacc(r, _):
            for q in range(WROW // 16):
                accL[r, pl.ds(q * 16, 16)] = zero16f
            return 0
        lax.fori_loop(0, LROWS, _zacc, 0)

        def _pfP(r, _):
            for q in range(128 // 16):
                csrcP[r, pl.ds(q * 16, 16)] = zero16i
                crelP[r, pl.ds(q * 16, 16)] = jnp.full((16,), R, jnp.int32)
            return 0
        lax.fori_loop(0, CBP // 128, _pfP, 0)

        def chunk_body(ci, k):
            eoff = sid * ET + ci * CH
            pltpu.sync_copy(src_hbm.at[pl.ds(eoff, CH)], sd_src)
            pltpu.sync_copy(dst_hbm.at[pl.ds(eoff, CH)], sd_dst)

            def vreg_body(vi, k):
                dv = sd_dst[pl.ds(vi * 16, 16)]
                sv = sd_src[pl.ds(vi * 16, 16)]
                rel = dv - lo
                m = (rel >= 0) & (rel < R)
                mi = m.astype(jnp.int32)
                pos = k + plsc.cumsum(mi) - 1
                m = m & (pos < CBP)
                plsc.store_scatter(csrcP, [pos >> 7, pos & 127], sv, mask=m)
                plsc.store_scatter(crelP, [pos >> 7, pos & 127], rel, mask=m)
                return k + jnp.sum(mi)

            return lax.fori_loop(0, NV, vreg_body, k)

        lax.fori_loop(0, ET // CH, chunk_body, jnp.int32(0))

        pltpu.sync_copy(csrcP, ssrc.at[sid])
        pltpu.sync_copy(crelP, srel.at[sid])
        pltpu.sync_copy(
            ad_hbm.at[pl.ds((lo + sid * STRIPE) * N_HEADS,
                            LROWS * N_HEADS)], adch)
        plsc.subcore_barrier()

        def _pf2(i, _):
            csrc2[pl.ds(i * 16, 16)] = zero16i
            crel2[pl.ds(i * 16, 16)] = jnp.full((16,), DUMMYL, jnp.int32)
            return 0
        lax.fori_loop(0, CB2 // 16, _pf2, 0)

        slo = sid * STRIPE

        def t_body(tp, k2):
            def c_body(c, k2):
                pltpu.sync_copy(ssrc.at[tp, pl.ds(c * 8, 8)], lbufS)
                pltpu.sync_copy(srel.at[tp, pl.ds(c * 8, 8)], lbufR)

                def v_body(v, k2):
                    row = v >> 3
                    colb = (v & 7) * 16
                    relv = lbufR[row, pl.ds(colb, 16)]
                    sv = lbufS[row, pl.ds(colb, 16)]
                    rloc = relv - slo
                    m = (rloc >= 0) & (rloc < STRIPE)
                    mi = m.astype(jnp.int32)
                    pos = k2 + plsc.cumsum(mi) - 1
                    m = m & (pos < CB2)
                    plsc.store_scatter(csrc2, [pos], sv, mask=m)
                    plsc.store_scatter(crel2, [pos], rloc, mask=m)
                    return k2 + jnp.sum(mi)

                return lax.fori_loop(0, 64, v_body, k2)
            return lax.fori_loop(0, CBP // 1024, c_body, k2)

        k2 = lax.fori_loop(0, N_TILES, t_body, jnp.int32(0))
        k2 = jnp.minimum(k2, CB2)

        ng = (k2 + G - 1) // G
        bufs = (hr0, hr1)
        sems = (sem0, sem1)

        def _start(g, buf, sem):
            pltpu.async_copy(h_hbm.at[csrc2.at[pl.ds(g * G, G)]], buf, sem)

        def _process(g, hrows):
            def scale_acc(e, _):
                es = jnp.full((16,), e, jnp.int32)
                rel = plsc.load_gather(crel2, [g * G + es])
                s = plsc.load_gather(hrows, [es, ASRC0 + (it & 3)])
                d = plsc.load_gather(adch, [rel * N_HEADS + (it & 3)])
                a = s + d
                pv = jnp.exp(jnp.maximum(a, 0.2 * a))
                hrows[e, pl.ds(HC, 16)] = pv
                plsc.addupdate_scatter(accL, [rel, HC + it], pv)
                for hh in range(N_HEADS):
                    sp = plsc.load_gather(
                        hrows, [es, jnp.full((16,), HC + hh, jnp.int32)])
                    for q in range(OUT_CH // 16):
                        col = hh * OUT_CH + q * 16
                        v = hrows[e, pl.ds(col, 16)] * sp
                        plsc.addupdate_scatter(accL, [rel, col + it], v)
                return 0
            lax.fori_loop(0, G, scale_acc, 0)

        for b in range(2):
            @pl.when(b < ng)
            def _(b=b):
                _start(b, bufs[b], sems[b])

        def gp_body(gp, _):
            for b in range(2):
                g = gp * 2 + b

                @pl.when(g < ng)
                def _(g=g, b=b):
                    pltpu.make_async_copy(
                        h_hbm.at[csrc2.at[pl.ds(g * G, G)]],
                        bufs[b], sems[b]).wait()
                    _process(g, bufs[b])

                    @pl.when(g + 2 < ng)
                    def _():
                        _start(g + 2, bufs[b], sems[b])
            return 0
        lax.fori_loop(0, (ng + 1) // 2, gp_body, 0)

        def norm_body(t, _):
            base = lo + slo + t * 8

            @pl.when(base < N_NODES)
            def _():
                def row_body(j, _):
                    jr = t * 8 + j
                    js = jnp.full((16,), jr, jnp.int32)
                    sp = []
                    for hh in range(N_HEADS):
                        denom = plsc.load_gather(
                            accL, [js, jnp.full((16,), HC + hh, jnp.int32)])
                        sp.append(1.0 / (denom + 1e-16))
                    for q in range(OUT_CH // 16):
                        acc_v = accL[jr, pl.ds(q * 16, 16)] * sp[0]
                        for hh in range(1, N_HEADS):
                            acc_v = acc_v + (
                                accL[jr, pl.ds(hh * OUT_CH + q * 16, 16)]
                                * sp[hh])
                        nout[j, pl.ds(q * 16, 16)] = (
                            acc_v * 0.25 + biasb[pl.ds(q * 16, 16)])
                    return 0
                lax.fori_loop(0, 8, row_body, 0)

                pltpu.sync_copy(nout, out_hbm.at[pl.ds(base, 8)])
            return 0
        lax.fori_loop(0, STRIPE // 8, norm_body, 0)
        plsc.subcore_barrier()
        return 0

    lax.fori_loop(0, NPASS, pass_body, 0)


def _sc_aggregate(h, ad, src, dst, bias):
    mesh = plsc.VectorSubcoreMesh(core_axis_name="c", subcore_axis_name="s")
    f32 = jnp.float32
    i32 = jnp.int32
    kern = pl.kernel(
        _sc_body,
        out_type=jax.ShapeDtypeStruct((N_NODES, OUT_CH), f32),
        mesh=mesh,
        compiler_params=pltpu.CompilerParams(needs_layout_passes=False),
        scratch_types=[
            pltpu.VMEM((CH,), i32),
            pltpu.VMEM((CH,), i32),
            pltpu.VMEM((CBP // 128, 128), i32),
            pltpu.VMEM((CBP // 128, 128), i32),
            pltpu.VMEM((8, 128), i32),
            pltpu.VMEM((8, 128), i32),
            pltpu.VMEM((CB2,), i32),
            pltpu.VMEM((CB2,), i32),
            pltpu.VMEM((LROWS * N_HEADS,), f32),
            pltpu.VMEM((G, WROW), f32),
            pltpu.VMEM((G, WROW), f32),
            pltpu.VMEM((LROWS, WROW), f32),
            pltpu.VMEM((8, OUT_CH), f32),
            pltpu.VMEM((OUT_CH,), f32),
            pltpu.VMEM_SHARED((N_TILES, CBP // 128, 128), i32),
            pltpu.VMEM_SHARED((N_TILES, CBP // 128, 128), i32),
            pltpu.SemaphoreType.DMA,
            pltpu.SemaphoreType.DMA,
        ],
    )
    return kern(h, ad, src, dst, bias)


def kernel(x, edge_index, W, att_src, att_dst, bias):
    b, s, n, f = x.shape
    xf = x.reshape(b * s * n, f)

    ei = edge_index.astype(jnp.int32)
    offs = jnp.arange(b * s, dtype=jnp.int32) * n
    src_r = (ei[0][None, :] + offs[:, None]).reshape(-1)
    dst_r = (ei[1][None, :] + offs[:, None]).reshape(-1)
    loop = jnp.arange(N_NODES, dtype=jnp.int32)
    pad = E_PAD - E_TOT
    src_all = jnp.concatenate(
        [src_r, loop, jnp.zeros((pad,), jnp.int32)])
    dst_all = jnp.concatenate(
        [dst_r, loop, jnp.full((pad,), jnp.int32(1 << 20))])
    src_all = src_all.reshape(ET, N_TILES).T.reshape(-1)
    dst_all = dst_all.reshape(ET, N_TILES).T.reshape(-1)

    h, ad = _tc_project(xf, W, att_src, att_dst)
    ad1 = jnp.pad(ad.reshape(-1),
                  (0, (NPASS * 2 * R + LROWS - N_NODES) * N_HEADS))
    out = _sc_aggregate(h, ad1, src_all, dst_all, bias)
    return out.reshape(b, s, n, OUT_CH)
